# baseline (speedup 1.0000x reference)
import functools

import jax
import jax.numpy as jnp
from jax.experimental import pallas as pl
from jax.experimental.pallas import tpu as pltpu

N = 10000
D = 128
NF = 128
C = 40


def _mm_kernel(x_ref, w_ref, b_ref, o_ref):
    o_ref[...] = jnp.dot(x_ref[...], w_ref[...],
                         preferred_element_type=jnp.float32) + b_ref[...]


def _matmul_bias(x, w, b):
    m, k = x.shape
    n = w.shape[1]
    bm = 512
    grid = (pl.cdiv(m, bm),)
    return pl.pallas_call(
        _mm_kernel,
        grid=grid,
        in_specs=[
            pl.BlockSpec((bm, k), lambda i: (i, 0)),
            pl.BlockSpec((k, n), lambda i: (0, 0)),
            pl.BlockSpec((1, n), lambda i: (0, 0)),
        ],
        out_specs=pl.BlockSpec((bm, n), lambda i: (i, 0)),
        out_shape=jax.ShapeDtypeStruct((m, n), jnp.float32),
    )(x, w, b.reshape(1, n))


def kernel(x, edge_index, edge_x, W1, b1, W2, b2, Wc1, bc1, Wc2, bc2):
    e = jax.nn.relu(_matmul_bias(edge_x, W1, b1))
    ew = jax.nn.sigmoid(_matmul_bias(e, W2, b2)).reshape(-1)

    loop = jnp.arange(N, dtype=edge_index.dtype)
    row = jnp.concatenate([edge_index[0], loop])
    col = jnp.concatenate([edge_index[1], loop])
    w = jnp.concatenate([ew, jnp.ones((N,), dtype=jnp.float32)])
    deg = jax.ops.segment_sum(w, col, num_segments=N)
    dinv = jnp.where(deg > 0, jax.lax.rsqrt(jnp.maximum(deg, 1e-12)), 0.0)
    norm = dinv[row] * w * dinv[col]

    h = _matmul_bias(x, Wc1, jnp.zeros_like(bc1))
    msg = h[row] * norm[:, None]
    h = jax.ops.segment_sum(msg, col, num_segments=N) + bc1
    h = jax.nn.relu(h)

    h2 = _matmul_bias(h, Wc2, jnp.zeros_like(bc2))
    msg = h2[row] * norm[:, None]
    out = jax.ops.segment_sum(msg, col, num_segments=N) + bc2
    return out


# speedup vs baseline: 9.8575x; 9.8575x over previous
import functools

import jax
import jax.numpy as jnp
from jax import lax
from jax.experimental import pallas as pl
from jax.experimental.pallas import tpu as pltpu
from jax.experimental.pallas import tpu_sc as plsc

N = 10000
NP = 10240
E = 320000
EP = 327680
D = 128
F2 = 64
C = 40
ECH = EP // 128

_NC, _NS = 2, 16


def _splat16(val_ref, idxs):
    return plsc.load_gather(val_ref, [jnp.full((16,), i, jnp.int32) for i in idxs])


def _edge_mlp_body(ex_ref, w1_ref, b1_ref, w2_ref, b2_ref, o_ref):
    e = jnp.dot(ex_ref[...], w1_ref[...], preferred_element_type=jnp.float32)
    e = jnp.maximum(e + b1_ref[...], 0.0)
    o = jnp.dot(e, w2_ref[...], preferred_element_type=jnp.float32) + b2_ref[...]
    o_ref[...] = jax.nn.sigmoid(o)


def _edge_mlp(edge_x, W1, b1, W2, b2):
    be = 6400
    return pl.pallas_call(
        _edge_mlp_body,
        grid=(E // be,),
        in_specs=[
            pl.BlockSpec((be, 16), lambda i: (i, 0)),
            pl.BlockSpec((16, 16), lambda i: (0, 0)),
            pl.BlockSpec((1, 16), lambda i: (0, 0)),
            pl.BlockSpec((16, 1), lambda i: (0, 0)),
            pl.BlockSpec((1, 1), lambda i: (0, 0)),
        ],
        out_specs=pl.BlockSpec((be, 1), lambda i: (i, 0)),
        out_shape=jax.ShapeDtypeStruct((E, 1), jnp.float32),
    )(edge_x, W1, b1.reshape(1, 16), W2, b2.reshape(1, 1))


def _xw1_body(x_ref, w_ref, o_ref):
    o_ref[...] = jnp.dot(x_ref[...], w_ref[0],
                         preferred_element_type=jnp.float32)[None]


def _xw1(xp, Wc1s):
    bn = 1024
    return pl.pallas_call(
        _xw1_body,
        grid=(NP // bn, 2),
        in_specs=[
            pl.BlockSpec((bn, D), lambda i, c: (i, 0)),
            pl.BlockSpec((1, D, D // 2), lambda i, c: (c, 0, 0)),
        ],
        out_specs=pl.BlockSpec((1, bn, D // 2), lambda i, c: (c, i, 0)),
        out_shape=jax.ShapeDtypeStruct((2, NP, D // 2), jnp.float32),
    )(xp, Wc1s)


def _scale1_body(da_ref, db_ref, x_ref, dinv_ref, xs_ref):
    d = lax.rsqrt(1.0 + da_ref[0] + db_ref[0])
    dinv_ref[...] = d
    xs_ref[0] = x_ref[0] * d


def _scale1(degAB3, X1):
    bn = 1024
    return pl.pallas_call(
        _scale1_body,
        grid=(NP // bn, 2),
        in_specs=[
            pl.BlockSpec((1, bn, 1), lambda i, c: (0, i, 0)),
            pl.BlockSpec((1, bn, 1), lambda i, c: (1, i, 0)),
            pl.BlockSpec((1, bn, D // 2), lambda i, c: (c, i, 0)),
        ],
        out_specs=[
            pl.BlockSpec((bn, 1), lambda i, c: (i, 0)),
            pl.BlockSpec((1, bn, D // 2), lambda i, c: (c, i, 0)),
        ],
        out_shape=[
            jax.ShapeDtypeStruct((NP, 1), jnp.float32),
            jax.ShapeDtypeStruct((2, NP, D // 2), jnp.float32),
        ],
    )(degAB3, degAB3, X1)


def _mm2_body(ha_ref, hb_ref, w_ref, dinv_ref, xs_ref):
    x2 = (jnp.dot(ha_ref[0], w_ref[0:64, :], preferred_element_type=jnp.float32)
          + jnp.dot(hb_ref[0], w_ref[64:128, :],
                    preferred_element_type=jnp.float32))
    xs_ref[...] = x2 * dinv_ref[...]


def _mm2(h1, Wc2p, dinv):
    bn = 1024
    return pl.pallas_call(
        _mm2_body,
        grid=(NP // bn,),
        in_specs=[
            pl.BlockSpec((1, bn, 64), lambda i: (0, i, 0)),
            pl.BlockSpec((1, bn, 64), lambda i: (1, i, 0)),
            pl.BlockSpec((D, F2), lambda i: (0, 0)),
            pl.BlockSpec((bn, 1), lambda i: (i, 0)),
        ],
        out_specs=pl.BlockSpec((bn, F2), lambda i: (i, 0)),
        out_shape=jax.ShapeDtypeStruct((NP, F2), jnp.float32),
    )(h1, h1, Wc2p, dinv)


def _sc_mesh():
    return plsc.VectorSubcoreMesh(core_axis_name="c", subcore_axis_name="s")


_SC_PARAMS = pltpu.CompilerParams(needs_layout_passes=False,
                                  use_tc_tiling_on_sc=False)


def _deg_body(col_hbm, ew_hbm, zeros_hbm, deg_out,
              deg_sh, cbuf, ebuf):
    c = lax.axis_index("c")
    s = lax.axis_index("s")
    w = c * _NS + s
    n0 = s * (NP // _NS)
    pltpu.sync_copy(zeros_hbm.at[pl.ds(n0, NP // _NS)],
                    deg_sh.at[pl.ds(n0, NP // _NS)])
    plsc.subcore_barrier()
    nrow = ECH // (_NC * _NS)

    def chunk(j, carry):
        pltpu.sync_copy(ebuf.at[j], deg_sh.at[cbuf.at[j]], add=True)
        return carry

    if _DIAG_SINGLE_TILE:
        @pl.when(s == 0)
        def _():
            def dgrp(g, carry):
                e0 = c * (ECH // _NC) + g * nrow
                pltpu.sync_copy(col_hbm.at[pl.ds(e0, nrow)], cbuf)
                pltpu.sync_copy(ew_hbm.at[pl.ds(e0, nrow)], ebuf)
                lax.fori_loop(0, nrow, chunk, 0)
                return carry

            lax.fori_loop(0, _NS, dgrp, 0)
    else:
        e0 = w * nrow
        pltpu.sync_copy(col_hbm.at[pl.ds(e0, nrow)], cbuf)
        pltpu.sync_copy(ew_hbm.at[pl.ds(e0, nrow)], ebuf)
        lax.fori_loop(0, nrow, chunk, 0)
    plsc.subcore_barrier()
    pltpu.sync_copy(deg_sh.at[pl.ds(n0, NP // _NS)],
                    deg_out.at[pl.ds(c * NP + n0, NP // _NS)])


def _deg_kernel(colp2, ewp2, zeros_np):
    k = pl.kernel(
        _deg_body,
        out_type=jax.ShapeDtypeStruct((2 * NP,), jnp.float32),
        mesh=_sc_mesh(),
        compiler_params=_SC_PARAMS,
        scratch_types=[
            pltpu.VMEM_SHARED((NP,), jnp.float32),
            pltpu.VMEM((ECH // 32, 128), jnp.int32),
            pltpu.VMEM((ECH // 32, 128), jnp.float32),
        ],
    )
    return k(colp2, ewp2, zeros_np)


def _spmm_body(F, row_hbm, col_hbm, ew_hbm, xs_hbm, dinv_hbm, b_hbm,
               out_hbm, xs_sh, acc_sh, rbuf, cbuf, ebuf, rows, obuf, dbuf,
               bbuf):
    nv = F // 16
    c = lax.axis_index("c")
    s = lax.axis_index("s")
    n0 = s * (NP // _NS)
    nblk = NP // _NS
    pltpu.sync_copy(xs_hbm.at[c, pl.ds(n0, nblk)], xs_sh.at[pl.ds(n0, nblk)])
    pltpu.sync_copy(xs_hbm.at[c, pl.ds(n0, nblk)], acc_sh.at[pl.ds(n0, nblk)])
    pltpu.sync_copy(dinv_hbm.at[pl.ds(n0, nblk)], dbuf)
    pltpu.sync_copy(b_hbm.at[c], bbuf)
    plsc.subcore_barrier()

    nch = ECH if _DIAG_SINGLE_TILE else ECH // _NS
    e0 = 0 if _DIAG_SINGLE_TILE else s * nch

    def group(g, carry):
        pltpu.sync_copy(row_hbm.at[pl.ds(e0 + g * _GRP, _GRP)], rbuf)
        pltpu.sync_copy(col_hbm.at[pl.ds(e0 + g * _GRP, _GRP)], cbuf)
        pltpu.sync_copy(ew_hbm.at[pl.ds(e0 + g * _GRP, _GRP)], ebuf)

        def chunk(j, carry2):
            pltpu.sync_copy(xs_sh.at[rbuf.at[j]], rows)

            def edge(i, cc):
                sp = _splat16(ebuf, (j, i))
                for f in range(nv):
                    v = rows[i, pl.ds(f * 16, 16)]
                    rows[i, pl.ds(f * 16, 16)] = v * sp
                return cc

            lax.fori_loop(0, 128, edge, 0)
            pltpu.sync_copy(rows, acc_sh.at[cbuf.at[j]], add=True)
            return carry2

        lax.fori_loop(0, _GRP, chunk, 0)
        return carry

    if _DIAG_SINGLE_TILE:
        @pl.when(s == 0)
        def _():
            lax.fori_loop(0, nch // _GRP, group, 0)
    else:
        lax.fori_loop(0, nch // _GRP, group, 0)
    plsc.subcore_barrier()

    def finblk(t, carry):
        b0 = t * _FB
        pltpu.sync_copy(acc_sh.at[pl.ds(n0 + b0, _FB)], obuf)

        def fin(i, cc):
            sd = _splat16(dbuf, (b0 + i, 0))
            for f in range(nv):
                v = obuf[i, pl.ds(f * 16, 16)]
                b = bbuf[0, pl.ds(f * 16, 16)]
                v = v * sd + b
                if F == 64:
                    v = jnp.maximum(v, 0.0)
                obuf[i, pl.ds(f * 16, 16)] = v
            return cc

        lax.fori_loop(0, _FB, fin, 0)
        pltpu.sync_copy(obuf, out_hbm.at[c, pl.ds(n0 + b0, _FB)])
        return carry

    lax.fori_loop(0, nblk // _FB, finblk, 0)


_GRP = 8
_FB = 64
_DIAG_SINGLE_TILE = False


def _spmm_kernel(F, rowp2, colp2, ewp2, Xs, dinv, bias):
    k = pl.kernel(
        functools.partial(_spmm_body, F),
        out_type=jax.ShapeDtypeStruct((2, NP, F), jnp.float32),
        mesh=_sc_mesh(),
        compiler_params=_SC_PARAMS,
        scratch_types=[
            pltpu.VMEM_SHARED((NP, F), jnp.float32),
            pltpu.VMEM_SHARED((NP, F), jnp.float32),
            pltpu.VMEM((_GRP, 128), jnp.int32),
            pltpu.VMEM((_GRP, 128), jnp.int32),
            pltpu.VMEM((_GRP, 128), jnp.float32),
            pltpu.VMEM((128, F), jnp.float32),
            pltpu.VMEM((_FB, F), jnp.float32),
            pltpu.VMEM((NP // _NS, 1), jnp.float32),
            pltpu.VMEM((1, F), jnp.float32),
        ],
    )
    return k(rowp2, colp2, ewp2, Xs, dinv, bias)


def kernel(x, edge_index, edge_x, W1, b1, W2, b2, Wc1, bc1, Wc2, bc2):
    xp = jnp.pad(x, ((0, NP - N), (0, 0)))
    rowp = jnp.pad(edge_index[0], (0, EP - E)).reshape(ECH, 128)
    colp = jnp.pad(edge_index[1], (0, EP - E)).reshape(ECH, 128)
    Wc2p = jnp.pad(Wc2, ((0, 0), (0, F2 - C)))
    bc2p = jnp.pad(bc2, (0, F2 - C))

    ew = _edge_mlp(edge_x, W1, b1, W2, b2)
    ewp = jnp.pad(ew.reshape(-1), (0, EP - E)).reshape(ECH, 128)

    Wc1s = Wc1.reshape(D, 2, D // 2).transpose(1, 0, 2)
    X1 = _xw1(xp, Wc1s)
    degAB = _deg_kernel(colp, ewp, jnp.zeros((NP,), jnp.float32))
    dinv, Xs1 = _scale1(degAB.reshape(2, NP, 1), X1)

    h1 = _spmm_kernel(64, rowp, colp, ewp, Xs1, dinv,
                      bc1.reshape(2, 1, 64))

    Xs2v = _mm2(h1, Wc2p, dinv)
    Xs2 = Xs2v.reshape(NP, 2, 32).transpose(1, 0, 2)

    out2 = _spmm_kernel(32, rowp, colp, ewp, Xs2, dinv,
                        bc2p.reshape(2, 1, 32))
    out = out2.transpose(1, 0, 2).reshape(NP, F2)
    return out[:N, :C]
